# baseline (speedup 1.0000x reference)
import jax
import jax.numpy as jnp
from jax.experimental import pallas as pl
from jax.experimental.pallas import tpu as pltpu
from jax.experimental.pallas import tpu_sc as plsc

EMB = 128
HID = 100
HPAD = 128
GW = 128


def _sc_gather_rows(emb, idx_flat):
    n = idx_flat.shape[0]
    mesh = plsc.VectorSubcoreMesh(core_axis_name="core",
                                  subcore_axis_name="subcore")

    @pl.kernel(out_type=jax.ShapeDtypeStruct((n, emb.shape[1]), emb.dtype),
               mesh=mesh)
    def gather_kernel(emb_hbm, idx_hbm, out_hbm):
        def body(i_vmem, o_vmem):
            pltpu.sync_copy(emb_hbm.at[i_vmem.at[0]], o_vmem)

        pltpu.emit_pipeline(
            body,
            grid=(n // GW,),
            in_specs=[pl.BlockSpec((1, GW), index_map=lambda i: (0, i))],
            out_specs=[pl.BlockSpec((GW, emb.shape[1]),
                                    index_map=lambda i: (i, 0))],
            core_axis_name=("core", "subcore"),
            dimension_semantics=(pltpu.PARALLEL,),
        )(idx_hbm, out_hbm)

    return gather_kernel(emb, idx_flat.reshape(1, n))


def _lstm_step(e_ref, W_ref, U_ref, b_ref, wd_ref, bd_ref, out_ref,
               h_ref, c_ref):
    t = pl.program_id(0)
    T = pl.num_programs(0)

    @pl.when(t == 0)
    def _():
        h_ref[...] = jnp.zeros_like(h_ref)
        c_ref[...] = jnp.zeros_like(c_ref)

    h = h_ref[...]
    z = jnp.dot(e_ref[...].astype(jnp.bfloat16), W_ref[...],
                preferred_element_type=jnp.float32)
    z = z + jnp.dot(h.astype(jnp.bfloat16), U_ref[...],
                    preferred_element_type=jnp.float32)
    z = z + b_ref[...]
    i = jax.nn.sigmoid(z[:, 0 * HPAD:1 * HPAD])
    f = jax.nn.sigmoid(z[:, 1 * HPAD:2 * HPAD])
    g = jnp.tanh(z[:, 2 * HPAD:3 * HPAD])
    o = jax.nn.sigmoid(z[:, 3 * HPAD:4 * HPAD])
    c = f * c_ref[...] + i * g
    h = o * jnp.tanh(c)
    c_ref[...] = c
    h_ref[...] = h

    @pl.when(t == T - 1)
    def _():
        logits = jnp.dot(h, wd_ref[...], preferred_element_type=jnp.float32)
        out_ref[...] = jax.nn.sigmoid(logits + bd_ref[...])


def _lstm_head(e, Wp, Up, bp, wdp, bdp, batch, seq):
    return pl.pallas_call(
        _lstm_step,
        grid=(seq,),
        in_specs=[
            pl.BlockSpec((batch, EMB), lambda t: (t, 0)),
            pl.BlockSpec((EMB, 4 * HPAD), lambda t: (0, 0)),
            pl.BlockSpec((HPAD, 4 * HPAD), lambda t: (0, 0)),
            pl.BlockSpec((1, 4 * HPAD), lambda t: (0, 0)),
            pl.BlockSpec((HPAD, HPAD), lambda t: (0, 0)),
            pl.BlockSpec((1, HPAD), lambda t: (0, 0)),
        ],
        out_specs=pl.BlockSpec((batch, HPAD), lambda t: (0, 0)),
        out_shape=jax.ShapeDtypeStruct((batch, HPAD), jnp.float32),
        scratch_shapes=[
            pltpu.VMEM((batch, HPAD), jnp.float32),
            pltpu.VMEM((batch, HPAD), jnp.float32),
        ],
    )(e, Wp, Up, bp, wdp, bdp)


def kernel(x, emb, W, U, b, Wd, bd):
    batch, seq = x.shape

    Wp = jnp.pad(W.reshape(EMB, 4, HID),
                 ((0, 0), (0, 0), (0, HPAD - HID))).reshape(
                     EMB, 4 * HPAD).astype(jnp.bfloat16)
    Up = jnp.pad(U.reshape(HID, 4, HID),
                 ((0, HPAD - HID), (0, 0), (0, HPAD - HID))).reshape(
                     HPAD, 4 * HPAD).astype(jnp.bfloat16)
    bp = jnp.pad(b.reshape(4, HID),
                 ((0, 0), (0, HPAD - HID))).reshape(1, 4 * HPAD)
    wdp = jnp.pad(Wd, ((0, HPAD - HID), (0, HPAD - 1)))
    bdp = jnp.broadcast_to(bd.reshape(1, 1), (1, HPAD))

    idx = x.T.reshape(-1)
    e = _sc_gather_rows(emb, idx)

    out_full = _lstm_head(e, Wp, Up, bp, wdp, bdp, batch, seq)
    return out_full[:, :1]


# speedup vs baseline: 1.1626x; 1.1626x over previous
import jax
import jax.numpy as jnp
from jax.experimental import pallas as pl
from jax.experimental.pallas import tpu as pltpu
from jax.experimental.pallas import tpu_sc as plsc

EMB = 128
HID = 100
HPAD = 128
GW = 128


def _sc_gather_rows(emb, idx_flat):
    n = idx_flat.shape[0]
    mesh = plsc.VectorSubcoreMesh(core_axis_name="core",
                                  subcore_axis_name="subcore")

    @pl.kernel(out_type=jax.ShapeDtypeStruct((n, emb.shape[1]), emb.dtype),
               mesh=mesh)
    def gather_kernel(emb_hbm, idx_hbm, out_hbm):
        def body(i_vmem, o_vmem):
            pltpu.sync_copy(emb_hbm.at[i_vmem.at[0]], o_vmem)

        pltpu.emit_pipeline(
            body,
            grid=(n // GW,),
            in_specs=[pl.BlockSpec((1, GW), index_map=lambda i: (0, i))],
            out_specs=[pl.BlockSpec((GW, emb.shape[1]),
                                    index_map=lambda i: (i, 0))],
            core_axis_name=("core", "subcore"),
            dimension_semantics=(pltpu.PARALLEL,),
        )(idx_hbm, out_hbm)

    return gather_kernel(emb, idx_flat.reshape(1, n))


def _sig(v):
    return 0.5 * jnp.tanh(0.5 * v) + 0.5


def _lstm_step(e_ref, W_ref, U_ref, b_ref, wd_ref, bd_ref, out_ref,
               h_ref, c_ref):
    t = pl.program_id(0)
    T = pl.num_programs(0)

    @pl.when(t == 0)
    def _():
        h_ref[...] = jnp.zeros_like(h_ref)
        c_ref[...] = jnp.zeros_like(c_ref)

    h = h_ref[...]
    z32 = jnp.dot(e_ref[...].astype(jnp.bfloat16), W_ref[...],
                  preferred_element_type=jnp.float32)
    z32 = z32 + jnp.dot(h, U_ref[...], preferred_element_type=jnp.float32)
    z = z32.astype(jnp.bfloat16) + b_ref[...]
    half = jnp.bfloat16(0.5)
    i = half * jnp.tanh(z[:, 0 * HPAD:1 * HPAD]) + half
    f = half * jnp.tanh(z[:, 1 * HPAD:2 * HPAD]) + half
    g = jnp.tanh(z[:, 2 * HPAD:3 * HPAD])
    o = half * jnp.tanh(z[:, 3 * HPAD:4 * HPAD]) + half
    c = f * c_ref[...] + i * g
    h = o * jnp.tanh(c)
    c_ref[...] = c
    h_ref[...] = h

    @pl.when(t == T - 1)
    def _():
        logits = jnp.dot(h, wd_ref[...], preferred_element_type=jnp.float32)
        out_ref[...] = _sig(logits + bd_ref[...])


def _lstm_head(e, Wp, Up, bp, wdp, bdp, batch, seq):
    return pl.pallas_call(
        _lstm_step,
        grid=(seq,),
        in_specs=[
            pl.BlockSpec((batch, EMB), lambda t: (t, 0)),
            pl.BlockSpec((EMB, 4 * HPAD), lambda t: (0, 0)),
            pl.BlockSpec((HPAD, 4 * HPAD), lambda t: (0, 0)),
            pl.BlockSpec((1, 4 * HPAD), lambda t: (0, 0)),
            pl.BlockSpec((HPAD, HPAD), lambda t: (0, 0)),
            pl.BlockSpec((1, HPAD), lambda t: (0, 0)),
        ],
        out_specs=pl.BlockSpec((batch, HPAD), lambda t: (0, 0)),
        out_shape=jax.ShapeDtypeStruct((batch, HPAD), jnp.float32),
        scratch_shapes=[
            pltpu.VMEM((batch, HPAD), jnp.bfloat16),
            pltpu.VMEM((batch, HPAD), jnp.bfloat16),
        ],
    )(e, Wp, Up, bp, wdp, bdp)


def kernel(x, emb, W, U, b, Wd, bd):
    batch, seq = x.shape

    gate_scale = jnp.array([0.5, 0.5, 1.0, 0.5],
                           jnp.float32).reshape(1, 4, 1)
    Wp = (jnp.pad(W.reshape(EMB, 4, HID),
                  ((0, 0), (0, 0), (0, HPAD - HID))) * gate_scale).reshape(
                      EMB, 4 * HPAD).astype(jnp.bfloat16)
    Up = (jnp.pad(U.reshape(HID, 4, HID),
                  ((0, HPAD - HID), (0, 0), (0, HPAD - HID)))
          * gate_scale).reshape(HPAD, 4 * HPAD).astype(jnp.bfloat16)
    bp = (jnp.pad(b.reshape(4, HID), ((0, 0), (0, HPAD - HID)))
          * gate_scale.reshape(4, 1)).reshape(1, 4 * HPAD).astype(
              jnp.bfloat16)
    wdp = jnp.pad(Wd, ((0, HPAD - HID), (0, HPAD - 1))).astype(jnp.bfloat16)
    bdp = jnp.broadcast_to(bd.reshape(1, 1), (1, HPAD))

    idx = x.T.reshape(-1)
    e = _sc_gather_rows(emb, idx)

    out_full = _lstm_head(e, Wp, Up, bp, wdp, bdp, batch, seq)
    return out_full[:, :1]


# speedup vs baseline: 1.4252x; 1.2259x over previous
import jax
import jax.numpy as jnp
from jax.experimental import pallas as pl
from jax.experimental.pallas import tpu as pltpu
from jax.experimental.pallas import tpu_sc as plsc

EMB = 128
HID = 100
HPAD = 128
GW = 128


def _sc_gather_rows(emb, idx_flat):
    n = idx_flat.shape[0]
    mesh = plsc.VectorSubcoreMesh(core_axis_name="core",
                                  subcore_axis_name="subcore")

    @pl.kernel(out_type=jax.ShapeDtypeStruct((n, emb.shape[1]), emb.dtype),
               mesh=mesh)
    def gather_kernel(emb_hbm, idx_hbm, out_hbm):
        def body(i_vmem, o_vmem):
            pltpu.sync_copy(emb_hbm.at[i_vmem.at[0]], o_vmem)

        pltpu.emit_pipeline(
            body,
            grid=(n // GW,),
            in_specs=[pl.BlockSpec((1, GW), index_map=lambda i: (0, i))],
            out_specs=[pl.BlockSpec((GW, emb.shape[1]),
                                    index_map=lambda i: (i, 0))],
            core_axis_name=("core", "subcore"),
            dimension_semantics=(pltpu.PARALLEL,),
        )(idx_hbm, out_hbm)

    return gather_kernel(emb, idx_flat.reshape(1, n))


def _sig(v):
    return 0.5 * jnp.tanh(0.5 * v) + 0.5


def _lstm_step(e_ref, WU_ref, b_ref, wd_ref, bd_ref, out_ref,
               eh_ref, c_ref):
    t = pl.program_id(0)
    T = pl.num_programs(0)

    @pl.when(t == 0)
    def _():
        eh_ref[:, HPAD:] = jnp.zeros_like(eh_ref[:, HPAD:])
        c_ref[...] = jnp.zeros_like(c_ref)

    eh_ref[:, :HPAD] = e_ref[...].astype(jnp.bfloat16)
    z32 = jnp.dot(eh_ref[...], WU_ref[...],
                  preferred_element_type=jnp.float32)
    z = z32.astype(jnp.bfloat16) + b_ref[...]
    half = jnp.bfloat16(0.5)
    i = half * jnp.tanh(z[:, 0 * HPAD:1 * HPAD]) + half
    f = half * jnp.tanh(z[:, 1 * HPAD:2 * HPAD]) + half
    g = jnp.tanh(z[:, 2 * HPAD:3 * HPAD])
    o = half * jnp.tanh(z[:, 3 * HPAD:4 * HPAD]) + half
    c = f * c_ref[...] + i * g
    h = o * jnp.tanh(c)
    c_ref[...] = c
    eh_ref[:, HPAD:] = h

    @pl.when(t == T - 1)
    def _():
        logits = jnp.dot(h, wd_ref[...], preferred_element_type=jnp.float32)
        out_ref[...] = _sig(logits + bd_ref[...])


def _lstm_head(e, WUp, bp, wdp, bdp, batch, seq):
    return pl.pallas_call(
        _lstm_step,
        grid=(seq,),
        in_specs=[
            pl.BlockSpec((batch, EMB), lambda t: (t, 0)),
            pl.BlockSpec((EMB + HPAD, 4 * HPAD), lambda t: (0, 0)),
            pl.BlockSpec((1, 4 * HPAD), lambda t: (0, 0)),
            pl.BlockSpec((HPAD, HPAD), lambda t: (0, 0)),
            pl.BlockSpec((1, HPAD), lambda t: (0, 0)),
        ],
        out_specs=pl.BlockSpec((batch, HPAD), lambda t: (0, 0)),
        out_shape=jax.ShapeDtypeStruct((batch, HPAD), jnp.float32),
        scratch_shapes=[
            pltpu.VMEM((batch, EMB + HPAD), jnp.bfloat16),
            pltpu.VMEM((batch, HPAD), jnp.bfloat16),
        ],
    )(e, WUp, bp, wdp, bdp)


def kernel(x, emb, W, U, b, Wd, bd):
    batch, seq = x.shape

    gate_scale = jnp.array([0.5, 0.5, 1.0, 0.5],
                           jnp.float32).reshape(1, 4, 1)
    Wp = (jnp.pad(W.reshape(EMB, 4, HID),
                  ((0, 0), (0, 0), (0, HPAD - HID))) * gate_scale).reshape(
                      EMB, 4 * HPAD).astype(jnp.bfloat16)
    Up = (jnp.pad(U.reshape(HID, 4, HID),
                  ((0, HPAD - HID), (0, 0), (0, HPAD - HID)))
          * gate_scale).reshape(HPAD, 4 * HPAD).astype(jnp.bfloat16)
    bp = (jnp.pad(b.reshape(4, HID), ((0, 0), (0, HPAD - HID)))
          * gate_scale.reshape(4, 1)).reshape(1, 4 * HPAD).astype(
              jnp.bfloat16)
    wdp = jnp.pad(Wd, ((0, HPAD - HID), (0, HPAD - 1))).astype(jnp.bfloat16)
    bdp = jnp.broadcast_to(bd.reshape(1, 1), (1, HPAD))

    idx = x.T.reshape(-1)
    e = _sc_gather_rows(emb, idx)

    WUp = jnp.concatenate([Wp, Up], axis=0)
    out_full = _lstm_head(e, WUp, bp, wdp, bdp, batch, seq)
    return out_full[:, :1]
